# baseline (speedup 1.0000x reference)
import functools

import jax
import jax.numpy as jnp
from jax import lax
from jax.experimental import pallas as pl
from jax.experimental.pallas import tpu as pltpu, tpu_sc as plsc

B_TOTAL = 4 * 8192
D_EMB = 64
NC, NS = 2, 16
NW = NC * NS
CHUNK = 128
B_PER_W = B_TOTAL // NW
N_CHUNKS = B_PER_W // CHUNK

_mesh = plsc.VectorSubcoreMesh(core_axis_name="c", subcore_axis_name="s")


@functools.partial(
    pl.kernel,
    mesh=_mesh,
    out_type=jax.ShapeDtypeStruct((4, 8192, D_EMB), jnp.float32),
    scratch_types=[
        pltpu.VMEM((N_CHUNKS, CHUNK), jnp.int32),
        pltpu.VMEM((B_PER_W, D_EMB), jnp.float32),
        pltpu.VMEM_SHARED((8192, D_EMB), jnp.float32),
        [pltpu.SemaphoreType.DMA] * N_CHUNKS,
        pltpu.SemaphoreType.DMA,
    ],
    compiler_params=pltpu.CompilerParams(use_tc_tiling_on_sc=False),
)
def _gather(idx_hbm, table_hbm, out_hbm, idx_v, rows_v, table_sp, gsems, ssem):
    s = lax.axis_index("s")
    wid = s * NC + lax.axis_index("c")
    batch = wid // 8
    row_off = (wid % 8) * B_PER_W
    rows_per_tile = 8192 // NS
    pltpu.sync_copy(
        table_hbm.at[pl.ds(s * rows_per_tile, rows_per_tile)],
        table_sp.at[pl.ds(s * rows_per_tile, rows_per_tile)],
    )
    idxh = pltpu.async_copy(idx_hbm.at[wid], idx_v, ssem)
    plsc.subcore_barrier()
    idxh.wait()
    gh = []
    for j in range(N_CHUNKS):
        gh.append(
            pltpu.async_copy(
                table_sp.at[idx_v.at[j]],
                rows_v.at[pl.ds(j * CHUNK, CHUNK)],
                gsems[j],
            )
        )
    sh = []
    for j in range(N_CHUNKS):
        gh[j].wait()
        sh.append(
            pltpu.async_copy(
                rows_v.at[pl.ds(j * CHUNK, CHUNK)],
                out_hbm.at[batch, pl.ds(row_off + j * CHUNK, CHUNK)],
                ssem,
            )
        )
    for h in sh:
        h.wait()


def kernel(x, table):
    idx = x.reshape(NW, N_CHUNKS, CHUNK)
    return _gather(idx, table)


# speedup vs baseline: 1.0049x; 1.0049x over previous
import functools

import jax
import jax.numpy as jnp
from jax import lax
from jax.experimental import pallas as pl
from jax.experimental.pallas import tpu as pltpu, tpu_sc as plsc

NC, NS = 2, 16
NW = NC * NS
NB = 4
T_TOT = 8192
D_EMB = 64
ND1 = D_EMB // 8
NT1 = T_TOT // 128
TG = 8

_mesh = plsc.VectorSubcoreMesh(core_axis_name="c", subcore_axis_name="s")


@functools.partial(
    pl.kernel,
    mesh=_mesh,
    out_type=jax.ShapeDtypeStruct((NB, ND1, NT1, 8, 128), jnp.float32),
    scratch_types=[
        pltpu.VMEM((NT1, 128), jnp.int32),
        pltpu.VMEM((NT1 * 8 * 128,), jnp.float32),
        pltpu.VMEM((2, TG, 8, 128), jnp.float32),
        pltpu.SemaphoreType.DMA,
        pltpu.SemaphoreType.DMA,
        [pltpu.SemaphoreType.DMA] * 2,
    ],
    compiler_params=pltpu.CompilerParams(
        use_tc_tiling_on_sc=False, needs_layout_passes=False
    ),
)
def _gather(x_ph, tbl_ph, out_ph, idx_v, tbl_v, bufs, tsem, isem, ssems):
    wid = lax.axis_index("s") * NC + lax.axis_index("c")
    b = wid // ND1
    d1 = wid % ND1
    th = pltpu.async_copy(tbl_ph.at[d1], tbl_v, tsem)
    ih = pltpu.async_copy(x_ph.at[:, b], idx_v, isem)
    th.wait()
    ih.wait()
    store_h = [None, None]
    for g in range(NT1 // TG):
        slot = g % 2
        if store_h[slot] is not None:
            store_h[slot].wait()
        for t1l in range(TG):
            t1 = g * TG + t1l

            def chunk(k, _, t1=t1, t1l=t1l, slot=slot):
                idx = idx_v[t1, pl.ds(k * 16, 16)]
                base = ((idx >> 7) << 10) + (idx & 127)
                for d0 in range(8):
                    v = plsc.load_gather(tbl_v, [base + (d0 * 128)])
                    bufs[slot, t1l, d0, pl.ds(k * 16, 16)] = v
                return 0

            lax.fori_loop(0, 8, chunk, 0)
        store_h[slot] = pltpu.async_copy(
            bufs.at[slot],
            out_ph.at[b, d1, pl.ds(g * TG, TG)],
            ssems[slot],
        )
    store_h[0].wait()
    store_h[1].wait()


def kernel(x, table):
    x_ph = x.reshape(NB, NT1, 128).transpose(1, 0, 2)
    tbl_ph = (
        table.T.reshape(ND1, 8, NT1, 128)
        .transpose(0, 2, 1, 3)
        .reshape(ND1, NT1 * 8 * 128)
    )
    res = _gather(x_ph, tbl_ph)
    return res.transpose(0, 2, 4, 1, 3).reshape(NB, T_TOT, D_EMB)
